# baseline (speedup 1.0000x reference)
import functools

import jax
import jax.numpy as jnp
from jax.experimental import pallas as pl
from jax.experimental.pallas import tpu as pltpu

K_TOP = 64
KEY_BLOCK = 2048


def _score_kernel(q_ref, k_ref, out_ref):
    out_ref[...] = jax.lax.dot_general(
        q_ref[...], k_ref[...],
        dimension_numbers=(((1,), (1,)), ((), ())),
        preferred_element_type=jnp.float32,
    )


def kernel(queries, keys, values):
    B, D = queries.shape
    N = keys.shape[0]
    n_blocks = pl.cdiv(N, KEY_BLOCK)

    scores = pl.pallas_call(
        _score_kernel,
        grid=(n_blocks,),
        in_specs=[
            pl.BlockSpec((B, D), lambda j: (0, 0)),
            pl.BlockSpec((KEY_BLOCK, D), lambda j: (j, 0)),
        ],
        out_specs=pl.BlockSpec((B, KEY_BLOCK), lambda j: (0, j)),
        out_shape=jax.ShapeDtypeStruct((B, N), jnp.float32),
    )(queries, keys)

    top_ks, top_k_inds = jax.lax.top_k(scores, K_TOP)
    weights = jax.nn.softmax(top_ks, axis=1)
    selected_values = jnp.take(values, top_k_inds, axis=0)
    weighted_selection = jnp.sum(selected_values * weights[:, :, None], axis=1)
    return (weighted_selection, top_k_inds, weights)
